# baseline (speedup 1.0000x reference)
import math
import jax
import jax.numpy as jnp
from jax import lax
from jax.experimental import pallas as pl
from jax.experimental.pallas import tpu as pltpu

_S = 64.0
_M = 0.5
_COS_M = math.cos(_M)
_SIN_M = math.sin(_M)
_THRESHOLD = math.cos(math.pi - _M)
_MM = math.sin(math.pi - _M) * _M

_B = 1024
_C = 100000
_CBJ = 2048
_NJ = (_C + _CBJ - 1) // _CBJ
_GR = 64


def _gather_body(lbl_ref, *refs):
    blks, out_ref, acc_ref = refs[:_GR], refs[_GR], refs[_GR + 1]
    i = pl.program_id(0)
    io0 = lax.broadcasted_iota(jnp.int32, (8, 128), 0)
    io1 = lax.broadcasted_iota(jnp.int32, (8, 128), 1)

    @pl.when(i == 0)
    def _init():
        acc_ref[:, :] = jnp.zeros_like(acc_ref)

    upd = jnp.zeros((8, 128), jnp.float32)
    for k in range(_GR):
        r = _GR * i + k
        lbl = lbl_ref[r]
        m = (io0 == lbl % 8) & (io1 == r % 128)
        val = jnp.sum(jnp.where(m, blks[k][:, :], 0.0))
        slot = (io0 == r // 128) & (io1 == r % 128)
        upd = upd + jnp.where(slot, val, 0.0)
    acc_ref[:, :] += upd

    @pl.when(i == _B // _GR - 1)
    def _finish():
        out_ref[:, :] = acc_ref[:, :]


def _gather_targets(logits_t, labels):
    itile = 128 // _GR
    specs = [
        pl.BlockSpec((8, 128),
                     lambda i, lbl, k=k: (lbl[_GR * i + k] // 8, i // itile))
        for k in range(_GR)
    ]
    out = pl.pallas_call(
        _gather_body,
        grid_spec=pltpu.PrefetchScalarGridSpec(
            num_scalar_prefetch=1,
            grid=(_B // _GR,),
            in_specs=specs,
            out_specs=pl.BlockSpec((8, 128), lambda i, lbl: (0, 0)),
            scratch_shapes=[pltpu.VMEM((8, 128), jnp.float32)],
        ),
        out_shape=jax.ShapeDtypeStruct((8, 128), jnp.float32),
    )(labels, *([logits_t] * _GR))
    return out.reshape(1, _B)


def _loss_body(tgt_ref, t_ref, x_ref, loss_ref, tnew_ref, acc_ref):
    j = pl.program_id(0)

    @pl.when(j == 0)
    def _prep():
        tnew_ref[0] = 0.01 * jnp.mean(tgt_ref[:, :]) + 0.99 * t_ref[0]
        acc_ref[:, :] = jnp.zeros_like(acc_ref)

    t_new = tnew_ref[0]
    tl = tgt_ref[:, :]
    sin_theta = jnp.sqrt(1.0 - tl * tl)
    ctm = tl * _COS_M - sin_theta * _SIN_M

    x = x_ref[:, :]
    xm = jnp.where(x > ctm, x * (t_new + x), x)
    e = jnp.exp(_S * xm)

    @pl.when(j < _NJ - 1)
    def _accum():
        acc_ref[:, :] += jnp.sum(e, axis=0, keepdims=True)

    @pl.when(j == _NJ - 1)
    def _finish():
        valid = (j * _CBJ + lax.broadcasted_iota(jnp.int32, e.shape, 0)) < _C
        tail = jnp.sum(jnp.where(valid, e, 0.0), axis=0, keepdims=True)
        final = jnp.where(tl > _THRESHOLD, ctm, tl - _MM)
        num = _S * final
        mod_tl = jnp.where(tl > ctm, tl * (t_new + tl), tl)
        corr = jnp.exp(num) - jnp.exp(_S * mod_tl)
        rowsum = acc_ref[:, :] + tail + corr
        loss_ref[0] = -jnp.mean(num - jnp.log(rowsum))


def _tc_loss(target, t, logits_t):
    return pl.pallas_call(
        _loss_body,
        grid=(_NJ,),
        in_specs=[
            pl.BlockSpec((1, _B), lambda j: (0, 0)),
            pl.BlockSpec(memory_space=pltpu.SMEM),
            pl.BlockSpec((_CBJ, _B), lambda j: (j, 0)),
        ],
        out_specs=pl.BlockSpec(memory_space=pltpu.SMEM),
        out_shape=jax.ShapeDtypeStruct((1,), jnp.float32),
        scratch_shapes=[
            pltpu.SMEM((1,), jnp.float32),
            pltpu.VMEM((1, _B), jnp.float32),
        ],
    )(target, t, logits_t)


def kernel(logits, labels, t):
    logits_t = logits.T
    target = _gather_targets(logits_t, labels)
    loss = _tc_loss(target, t, logits_t)
    return loss[0]


# speedup vs baseline: 1.0091x; 1.0091x over previous
import math
import jax
import jax.numpy as jnp
from jax import lax
from jax.experimental import pallas as pl
from jax.experimental.pallas import tpu as pltpu

_S = 64.0
_M = 0.5
_COS_M = math.cos(_M)
_SIN_M = math.sin(_M)
_THRESHOLD = math.cos(math.pi - _M)
_MM = math.sin(math.pi - _M) * _M

_B = 1024
_C = 100000
_CBJ = 2048
_NJ = (_C + _CBJ - 1) // _CBJ
_GR = 32


def _gather_body(lbl_ref, *refs):
    blks, out_ref, acc_ref = refs[:_GR], refs[_GR], refs[_GR + 1]
    i = pl.program_id(0)
    io0 = lax.broadcasted_iota(jnp.int32, (8, 128), 0)
    io1 = lax.broadcasted_iota(jnp.int32, (8, 128), 1)

    @pl.when(i == 0)
    def _init():
        acc_ref[:, :] = jnp.zeros_like(acc_ref)

    upd = jnp.zeros((8, 128), jnp.float32)
    for k in range(_GR):
        r = _GR * i + k
        lbl = lbl_ref[r]
        m = (io0 == lbl % 8) & (io1 == r % 128)
        val = jnp.sum(jnp.where(m, blks[k][:, :], 0.0))
        slot = (io0 == r // 128) & (io1 == r % 128)
        upd = upd + jnp.where(slot, val, 0.0)
    acc_ref[:, :] += upd

    @pl.when(i == _B // _GR - 1)
    def _finish():
        out_ref[:, :] = acc_ref[:, :]


def _gather_targets(logits_t, labels):
    itile = 128 // _GR
    specs = [
        pl.BlockSpec((8, 128),
                     lambda i, lbl, k=k: (lbl[_GR * i + k] // 8, i // itile))
        for k in range(_GR)
    ]
    out = pl.pallas_call(
        _gather_body,
        grid_spec=pltpu.PrefetchScalarGridSpec(
            num_scalar_prefetch=1,
            grid=(_B // _GR,),
            in_specs=specs,
            out_specs=pl.BlockSpec((8, 128), lambda i, lbl: (0, 0)),
            scratch_shapes=[pltpu.VMEM((8, 128), jnp.float32)],
        ),
        out_shape=jax.ShapeDtypeStruct((8, 128), jnp.float32),
    )(labels, *([logits_t] * _GR))
    return out.reshape(1, _B)


def _loss_body(tgt_ref, t_ref, x_ref, loss_ref, tnew_ref, acc_ref):
    j = pl.program_id(0)

    @pl.when(j == 0)
    def _prep():
        tnew_ref[0] = 0.01 * jnp.mean(tgt_ref[:, :]) + 0.99 * t_ref[0]
        acc_ref[:, :] = jnp.zeros_like(acc_ref)

    t_new = tnew_ref[0]
    tl = tgt_ref[:, :]
    sin_theta = jnp.sqrt(1.0 - tl * tl)
    ctm = tl * _COS_M - sin_theta * _SIN_M

    x = x_ref[:, :]
    xm = jnp.where(x > ctm, x * (t_new + x), x)
    e = jnp.exp(_S * xm)

    @pl.when(j < _NJ - 1)
    def _accum():
        acc_ref[:, :] += jnp.sum(e, axis=0, keepdims=True)

    @pl.when(j == _NJ - 1)
    def _finish():
        valid = (j * _CBJ + lax.broadcasted_iota(jnp.int32, e.shape, 0)) < _C
        tail = jnp.sum(jnp.where(valid, e, 0.0), axis=0, keepdims=True)
        final = jnp.where(tl > _THRESHOLD, ctm, tl - _MM)
        num = _S * final
        mod_tl = jnp.where(tl > ctm, tl * (t_new + tl), tl)
        corr = jnp.exp(num) - jnp.exp(_S * mod_tl)
        rowsum = acc_ref[:, :] + tail + corr
        loss_ref[0] = -jnp.mean(num - jnp.log(rowsum))


def _tc_loss(target, t, logits_t):
    return pl.pallas_call(
        _loss_body,
        grid=(_NJ,),
        in_specs=[
            pl.BlockSpec((1, _B), lambda j: (0, 0)),
            pl.BlockSpec(memory_space=pltpu.SMEM),
            pl.BlockSpec((_CBJ, _B), lambda j: (j, 0)),
        ],
        out_specs=pl.BlockSpec(memory_space=pltpu.SMEM),
        out_shape=jax.ShapeDtypeStruct((1,), jnp.float32),
        scratch_shapes=[
            pltpu.SMEM((1,), jnp.float32),
            pltpu.VMEM((1, _B), jnp.float32),
        ],
    )(target, t, logits_t)


def kernel(logits, labels, t):
    logits_t = logits.T
    target = _gather_targets(logits_t, labels)
    loss = _tc_loss(target, t, logits_t)
    return loss[0]


# speedup vs baseline: 1.0346x; 1.0252x over previous
import math
import jax
import jax.numpy as jnp
from jax import lax
from jax.experimental import pallas as pl
from jax.experimental.pallas import tpu as pltpu

_S = 64.0
_M = 0.5
_COS_M = math.cos(_M)
_SIN_M = math.sin(_M)
_THRESHOLD = math.cos(math.pi - _M)
_MM = math.sin(math.pi - _M) * _M

_B = 1024
_C = 100000
_CBJ = 3072
_NJ = (_C + _CBJ - 1) // _CBJ
_GR = 32


def _gather_body(lbl_ref, *refs):
    blks, out_ref, acc_ref = refs[:_GR], refs[_GR], refs[_GR + 1]
    i = pl.program_id(0)
    io0 = lax.broadcasted_iota(jnp.int32, (8, 128), 0)
    io1 = lax.broadcasted_iota(jnp.int32, (8, 128), 1)

    @pl.when(i == 0)
    def _init():
        acc_ref[:, :] = jnp.zeros_like(acc_ref)

    upd = jnp.zeros((8, 128), jnp.float32)
    for k in range(_GR):
        r = _GR * i + k
        lbl = lbl_ref[r]
        m = (io0 == lbl % 8) & (io1 == r % 128)
        val = jnp.sum(jnp.where(m, blks[k][:, :], 0.0))
        slot = (io0 == r // 128) & (io1 == r % 128)
        upd = upd + jnp.where(slot, val, 0.0)
    acc_ref[:, :] += upd

    @pl.when(i == _B // _GR - 1)
    def _finish():
        out_ref[:, :] = acc_ref[:, :]


def _gather_targets(logits_t, labels):
    itile = 128 // _GR
    specs = [
        pl.BlockSpec((8, 128),
                     lambda i, lbl, k=k: (lbl[_GR * i + k] // 8, i // itile))
        for k in range(_GR)
    ]
    out = pl.pallas_call(
        _gather_body,
        grid_spec=pltpu.PrefetchScalarGridSpec(
            num_scalar_prefetch=1,
            grid=(_B // _GR,),
            in_specs=specs,
            out_specs=pl.BlockSpec((8, 128), lambda i, lbl: (0, 0)),
            scratch_shapes=[pltpu.VMEM((8, 128), jnp.float32)],
        ),
        out_shape=jax.ShapeDtypeStruct((8, 128), jnp.float32),
    )(labels, *([logits_t] * _GR))
    return out.reshape(1, _B)


def _loss_body(tgt_ref, t_ref, x_ref, loss_ref, tnew_ref, acc_ref):
    j = pl.program_id(0)

    @pl.when(j == 0)
    def _prep():
        tnew_ref[0] = 0.01 * jnp.mean(tgt_ref[:, :]) + 0.99 * t_ref[0]
        acc_ref[:, :] = jnp.zeros_like(acc_ref)

    t_new = tnew_ref[0]
    tl = tgt_ref[:, :]
    sin_theta = jnp.sqrt(1.0 - tl * tl)
    ctm = tl * _COS_M - sin_theta * _SIN_M

    x = x_ref[:, :]
    xm = jnp.where(x > ctm, x * (t_new + x), x)
    e = jnp.exp(_S * xm)

    @pl.when(j < _NJ - 1)
    def _accum():
        acc_ref[:, :] += jnp.sum(e, axis=0, keepdims=True)

    @pl.when(j == _NJ - 1)
    def _finish():
        valid = (j * _CBJ + lax.broadcasted_iota(jnp.int32, e.shape, 0)) < _C
        tail = jnp.sum(jnp.where(valid, e, 0.0), axis=0, keepdims=True)
        final = jnp.where(tl > _THRESHOLD, ctm, tl - _MM)
        num = _S * final
        mod_tl = jnp.where(tl > ctm, tl * (t_new + tl), tl)
        corr = jnp.exp(num) - jnp.exp(_S * mod_tl)
        rowsum = acc_ref[:, :] + tail + corr
        loss_ref[0] = -jnp.mean(num - jnp.log(rowsum))


def _tc_loss(target, t, logits_t):
    return pl.pallas_call(
        _loss_body,
        grid=(_NJ,),
        in_specs=[
            pl.BlockSpec((1, _B), lambda j: (0, 0)),
            pl.BlockSpec(memory_space=pltpu.SMEM),
            pl.BlockSpec((_CBJ, _B), lambda j: (j, 0)),
        ],
        out_specs=pl.BlockSpec(memory_space=pltpu.SMEM),
        out_shape=jax.ShapeDtypeStruct((1,), jnp.float32),
        scratch_shapes=[
            pltpu.SMEM((1,), jnp.float32),
            pltpu.VMEM((1, _B), jnp.float32),
        ],
    )(target, t, logits_t)


def kernel(logits, labels, t):
    logits_t = logits.T
    target = _gather_targets(logits_t, labels)
    loss = _tc_loss(target, t, logits_t)
    return loss[0]


# speedup vs baseline: 1.0400x; 1.0053x over previous
import math
import jax
import jax.numpy as jnp
from jax import lax
from jax.experimental import pallas as pl
from jax.experimental.pallas import tpu as pltpu

_S = 64.0
_M = 0.5
_COS_M = math.cos(_M)
_SIN_M = math.sin(_M)
_THRESHOLD = math.cos(math.pi - _M)
_MM = math.sin(math.pi - _M) * _M

_B = 1024
_C = 100000
_CBJ = 3584
_NJ = (_C + _CBJ - 1) // _CBJ
_GR = 32


def _gather_body(lbl_ref, *refs):
    blks, out_ref, acc_ref = refs[:_GR], refs[_GR], refs[_GR + 1]
    i = pl.program_id(0)
    io0 = lax.broadcasted_iota(jnp.int32, (8, 128), 0)
    io1 = lax.broadcasted_iota(jnp.int32, (8, 128), 1)

    @pl.when(i == 0)
    def _init():
        acc_ref[:, :] = jnp.zeros_like(acc_ref)

    upd = jnp.zeros((8, 128), jnp.float32)
    for k in range(_GR):
        r = _GR * i + k
        lbl = lbl_ref[r]
        m = (io0 == lbl % 8) & (io1 == r % 128)
        val = jnp.sum(jnp.where(m, blks[k][:, :], 0.0))
        slot = (io0 == r // 128) & (io1 == r % 128)
        upd = upd + jnp.where(slot, val, 0.0)
    acc_ref[:, :] += upd

    @pl.when(i == _B // _GR - 1)
    def _finish():
        out_ref[:, :] = acc_ref[:, :]


def _gather_targets(logits_t, labels):
    itile = 128 // _GR
    specs = [
        pl.BlockSpec((8, 128),
                     lambda i, lbl, k=k: (lbl[_GR * i + k] // 8, i // itile))
        for k in range(_GR)
    ]
    out = pl.pallas_call(
        _gather_body,
        grid_spec=pltpu.PrefetchScalarGridSpec(
            num_scalar_prefetch=1,
            grid=(_B // _GR,),
            in_specs=specs,
            out_specs=pl.BlockSpec((8, 128), lambda i, lbl: (0, 0)),
            scratch_shapes=[pltpu.VMEM((8, 128), jnp.float32)],
        ),
        out_shape=jax.ShapeDtypeStruct((8, 128), jnp.float32),
    )(labels, *([logits_t] * _GR))
    return out.reshape(1, _B)


def _loss_body(tgt_ref, t_ref, x_ref, loss_ref, tnew_ref, acc_ref):
    j = pl.program_id(0)

    @pl.when(j == 0)
    def _prep():
        tnew_ref[0] = 0.01 * jnp.mean(tgt_ref[:, :]) + 0.99 * t_ref[0]
        acc_ref[:, :] = jnp.zeros_like(acc_ref)

    t_new = tnew_ref[0]
    tl = tgt_ref[:, :]
    sin_theta = jnp.sqrt(1.0 - tl * tl)
    ctm = tl * _COS_M - sin_theta * _SIN_M

    x = x_ref[:, :]
    xm = jnp.where(x > ctm, x * (t_new + x), x)
    e = jnp.exp(_S * xm)

    @pl.when(j < _NJ - 1)
    def _accum():
        acc_ref[:, :] += jnp.sum(e, axis=0, keepdims=True)

    @pl.when(j == _NJ - 1)
    def _finish():
        valid = (j * _CBJ + lax.broadcasted_iota(jnp.int32, e.shape, 0)) < _C
        tail = jnp.sum(jnp.where(valid, e, 0.0), axis=0, keepdims=True)
        final = jnp.where(tl > _THRESHOLD, ctm, tl - _MM)
        num = _S * final
        mod_tl = jnp.where(tl > ctm, tl * (t_new + tl), tl)
        corr = jnp.exp(num) - jnp.exp(_S * mod_tl)
        rowsum = acc_ref[:, :] + tail + corr
        loss_ref[0] = -jnp.mean(num - jnp.log(rowsum))


def _tc_loss(target, t, logits_t):
    return pl.pallas_call(
        _loss_body,
        grid=(_NJ,),
        in_specs=[
            pl.BlockSpec((1, _B), lambda j: (0, 0)),
            pl.BlockSpec(memory_space=pltpu.SMEM),
            pl.BlockSpec((_CBJ, _B), lambda j: (j, 0)),
        ],
        out_specs=pl.BlockSpec(memory_space=pltpu.SMEM),
        out_shape=jax.ShapeDtypeStruct((1,), jnp.float32),
        scratch_shapes=[
            pltpu.SMEM((1,), jnp.float32),
            pltpu.VMEM((1, _B), jnp.float32),
        ],
    )(target, t, logits_t)


def kernel(logits, labels, t):
    logits_t = logits.T
    target = _gather_targets(logits_t, labels)
    loss = _tc_loss(target, t, logits_t)
    return loss[0]
